# baseline (speedup 1.0000x reference)
import jax
import jax.numpy as jnp
from jax.experimental import pallas as pl
import jax.experimental.pallas.tpu as pltpu

ROWS = 128
VOCAB = 100000
W = 128
CHUNKS_PER_BLK = 64
BLK = W * CHUNKS_PER_BLK
NUM_BLOCKS = -(-VOCAB // BLK)
TAIL_COLS = VOCAB - (NUM_BLOCKS - 1) * BLK
TAIL_FULL_CHUNKS = TAIL_COLS // W
TAIL_REM = TAIL_COLS - TAIL_FULL_CHUNKS * W


def _fold_loop(x_ref, vmax, vchunk, n_chunks, i, k0=0):
    def body(k, carry):
        vmax, vchunk = carry
        chunk = x_ref[:, pl.ds(k * W, W)]
        better = chunk > vmax
        vchunk = jnp.where(better, i * CHUNKS_PER_BLK + k, vchunk)
        vmax = jnp.where(better, chunk, vmax)
        return vmax, vchunk

    return jax.lax.fori_loop(k0, n_chunks, body, (vmax, vchunk))


def _argmax_body(x_ref, out_ref, vmax_ref, vchunk_ref):
    i = pl.program_id(0)

    @pl.when(i == 0)
    def _():
        vmax = x_ref[:, :W]
        vchunk = jnp.zeros((ROWS, W), jnp.int32)
        vmax, vchunk = _fold_loop(x_ref, vmax, vchunk, CHUNKS_PER_BLK, 0,
                                  k0=1)
        vmax_ref[...] = vmax
        vchunk_ref[...] = vchunk

    @pl.when(jnp.logical_and(i > 0, i < NUM_BLOCKS - 1))
    def _():
        vmax, vchunk = _fold_loop(x_ref, vmax_ref[...], vchunk_ref[...],
                                  CHUNKS_PER_BLK, i)
        vmax_ref[...] = vmax
        vchunk_ref[...] = vchunk

    @pl.when(i == NUM_BLOCKS - 1)
    def _():
        last = NUM_BLOCKS - 1
        vmax, vchunk = _fold_loop(x_ref, vmax_ref[...], vchunk_ref[...],
                                  TAIL_FULL_CHUNKS, last)
        if TAIL_REM:
            k = TAIL_FULL_CHUNKS
            chunk = x_ref[:, k * W:(k + 1) * W]
            col = jax.lax.broadcasted_iota(jnp.int32, (ROWS, W), 1)
            chunk = jnp.where(col < TAIL_REM, chunk, -jnp.inf)
            better = chunk > vmax
            vchunk = jnp.where(better, last * CHUNKS_PER_BLK + k, vchunk)
            vmax = jnp.where(better, chunk, vmax)

        m = jnp.max(vmax, axis=1, keepdims=True)
        slot = jax.lax.broadcasted_iota(jnp.int32, (ROWS, W), 1)
        gcol = vchunk * W + slot
        cand = jnp.where(vmax == m, gcol, jnp.int32(2**31 - 1))
        out_ref[...] = jnp.min(cand, axis=1, keepdims=True)


@jax.jit
def _argmax_pallas(m_logits):
    return pl.pallas_call(
        _argmax_body,
        grid=(NUM_BLOCKS,),
        in_specs=[pl.BlockSpec((ROWS, BLK), lambda i: (0, i))],
        out_specs=pl.BlockSpec((ROWS, 1), lambda i: (0, 0)),
        out_shape=jax.ShapeDtypeStruct((ROWS, 1), jnp.int32),
        scratch_shapes=[
            pltpu.VMEM((ROWS, W), jnp.float32),
            pltpu.VMEM((ROWS, W), jnp.int32),
        ],
    )(m_logits)


def kernel(m_logits):
    token = _argmax_pallas(m_logits.astype(jnp.float32))
    return token.astype(jnp.int64)


# speedup vs baseline: 1.0108x; 1.0108x over previous
import jax
import jax.numpy as jnp
from jax.experimental import pallas as pl
import jax.experimental.pallas.tpu as pltpu

ROWS = 128
VOCAB = 100000
RB = 16
W = 1024
NCHUNK = VOCAB // W
REM = VOCAB - NCHUNK * W


def _argmax_body(x_ref, out_ref):
    def body(k, carry):
        vmax, vchunk = carry
        chunk = x_ref[:, pl.ds(k * W, W)]
        better = chunk > vmax
        vchunk = jnp.where(better, k, vchunk)
        vmax = jnp.where(better, chunk, vmax)
        return vmax, vchunk

    vmax = x_ref[:, :W]
    vchunk = jnp.zeros((RB, W), jnp.int32)
    vmax, vchunk = jax.lax.fori_loop(1, NCHUNK, body, (vmax, vchunk))

    col = jax.lax.broadcasted_iota(jnp.int32, (RB, W), 1)
    chunk = x_ref[:, NCHUNK * W:(NCHUNK + 1) * W]
    chunk = jnp.where(col < REM, chunk, -jnp.inf)
    better = chunk > vmax
    vchunk = jnp.where(better, NCHUNK, vchunk)
    vmax = jnp.where(better, chunk, vmax)

    m = jnp.max(vmax, axis=1, keepdims=True)
    gcol = vchunk * W + col
    cand = jnp.where(vmax == m, gcol, jnp.int32(2**31 - 1))
    out_ref[...] = jnp.min(cand, axis=1, keepdims=True)


@jax.jit
def _argmax_pallas(m_logits):
    pad = (NCHUNK + 1) * W - VOCAB
    return pl.pallas_call(
        _argmax_body,
        grid=(ROWS // RB,),
        in_specs=[pl.BlockSpec((RB, (NCHUNK + 1) * W), lambda i: (i, 0))],
        out_specs=pl.BlockSpec((RB, 1), lambda i: (i, 0)),
        out_shape=jax.ShapeDtypeStruct((ROWS, 1), jnp.int32),
    )(m_logits)


def kernel(m_logits):
    token = _argmax_pallas(m_logits.astype(jnp.float32))
    return token.astype(jnp.int64)
